# baseline (speedup 1.0000x reference)
import functools

import jax
import jax.numpy as jnp
from jax import lax
from jax.experimental import pallas as pl
from jax.experimental.pallas import tpu as pltpu
from jax.experimental.pallas import tpu_sc as plsc

N_USERS = 25000
N_ITEMS = 25000
D = 64
N_LAYERS = 2
BATCH = 2048

NC = 2
NS = 16
L = 16
CHUNK = 128
NBUF = 3
G = 6
N_PAD = 25088
SLAB = N_PAD // NS


def _propagate(zeros, ego_u, ego_i, edges, nct):
    mesh = plsc.VectorSubcoreMesh(core_axis_name="c", subcore_axis_name="s")

    @functools.partial(
        pl.kernel,
        out_type=(jax.ShapeDtypeStruct((N_PAD, D), jnp.float32),
                  jax.ShapeDtypeStruct((N_PAD, D), jnp.float32)),
        mesh=mesh,
        scratch_types=[
            pltpu.VMEM((G, 3, CHUNK), jnp.int32),
            pltpu.VMEM((G, 3, CHUNK), jnp.int32),
            pltpu.VMEM((CHUNK, D), jnp.float32),
            pltpu.VMEM((CHUNK, D), jnp.float32),
            pltpu.VMEM((CHUNK, D), jnp.float32),
            pltpu.VMEM_SHARED((N_PAD, D), jnp.float32),
            pltpu.SemaphoreType.DMA,
            pltpu.SemaphoreType.DMA,
            pltpu.SemaphoreType.DMA,
            pltpu.SemaphoreType.DMA,
            pltpu.SemaphoreType.DMA,
            pltpu.SemaphoreType.DMA,
            pltpu.SemaphoreType.DMA,
            pltpu.SemaphoreType.DMA,
        ],
        compiler_params=pltpu.CompilerParams(needs_layout_passes=False,
                                             use_tc_tiling_on_sc=False),
    )
    def layer(zeros_hbm, ego_u_hbm, ego_i_hbm, edg_hbm,
              out_u, out_i, ibA, ibB, g0, g1, g2, acc,
              sg0, sg1, sg2, ss0, ss1, ss2, si0, si1):
        cid = lax.axis_index("c")
        sid = lax.axis_index("s")
        ibs = (ibA, ibB)
        isems = (si0, si1)
        gbufs = (g0, g1, g2)
        gsems = (sg0, sg1, sg2)
        ssems = (ss0, ss1, ss2)

        pltpu.sync_copy(zeros_hbm, acc.at[pl.ds(sid * SLAB, SLAB)])
        plsc.subcore_barrier()

        ngr = nct // G
        nbody = nct // (2 * G)

        def side(flip, table):
            dr, sr = (0, 1) if flip == 0 else (1, 0)

            def sel(t):
                tb = 0 if (t % (2 * G)) < G else 1
                return tb, t % G, t % NBUF

            def g_start(t):
                tb, u, k = sel(t)
                pltpu.async_copy(table.at[ibs[tb].at[u, sr]], gbufs[k],
                                 gsems[k])

            def g_wait(t):
                tb, u, k = sel(t)
                pltpu.make_async_copy(table.at[ibs[tb].at[u, sr]], gbufs[k],
                                      gsems[k]).wait()

            def s_start(t):
                tb, u, k = sel(t)
                pltpu.async_copy(gbufs[k], acc.at[ibs[tb].at[u, dr]],
                                 ssems[k], add=True)

            def s_wait(t):
                tb, u, k = sel(t)
                pltpu.make_async_copy(gbufs[k], acc.at[ibs[tb].at[u, dr]],
                                      ssems[k]).wait()

            def i_start(tb, g):
                pltpu.async_copy(edg_hbm.at[sid, pl.ds(g * G, G)], ibs[tb],
                                 isems[tb])

            def i_wait(tb):
                pltpu.make_async_copy(edg_hbm.at[sid, pl.ds(0, G)], ibs[tb],
                                      isems[tb]).wait()

            def scale(t):
                tb, u, k = sel(t)
                buf = gbufs[k]

                def body(e, _):
                    vv = plsc.bitcast(
                        plsc.load_gather(
                            ibs[tb], [jnp.full((L,), u, jnp.int32),
                                      jnp.full((L,), 2, jnp.int32),
                                      jnp.full((L,), e, jnp.int32)]),
                        jnp.float32)
                    for q in range(D // L):
                        sl = pl.ds(q * L, L)
                        buf[e, sl] = buf[e, sl] * vv
                    return 0

                lax.fori_loop(0, CHUNK, body, 0, unroll=4)

            pltpu.sync_copy(edg_hbm.at[sid, pl.ds(0, G)], ibA)
            g_start(0)
            g_start(1)

            def body(j, _):
                for t in range(2 * G):
                    g_wait(t)
                    scale(t)
                    s_start(t)
                    if t == 1:
                        i_start(1, 2 * j + 1)
                    if t == G + 1:
                        @pl.when(2 * j + 2 < ngr)
                        def _():
                            i_start(0, 2 * j + 2)
                    if t == 0:
                        @pl.when(j > 0)
                        def _():
                            s_wait(t - 1)
                    else:
                        s_wait(t - 1)
                    if t == G - 2:
                        i_wait(1)
                    if t == 2 * G - 2:
                        @pl.when(2 * j + 2 < ngr)
                        def _():
                            i_wait(0)
                    if t < 2 * G - 2:
                        g_start(t + 2)
                    else:
                        @pl.when(2 * j + 2 < ngr)
                        def _():
                            g_start(t + 2)
                return 0

            lax.fori_loop(0, nbody, body, 0)
            s_wait(2 * G - 1)

        @pl.when(cid == 0)
        def _():
            side(0, ego_i_hbm)

        @pl.when(cid == 1)
        def _():
            side(1, ego_u_hbm)

        plsc.subcore_barrier()
        sl = pl.ds(sid * SLAB, SLAB)

        @pl.when(cid == 0)
        def _():
            pltpu.sync_copy(acc.at[sl], out_u.at[sl])

        @pl.when(cid == 1)
        def _():
            pltpu.sync_copy(acc.at[sl], out_i.at[sl])

    return layer(zeros, ego_u, ego_i, edges)


def _finalize(u0, u1, u2, i0, i1, i2, users, pos_items):
    rows = BATCH // NS
    mesh = plsc.VectorSubcoreMesh(core_axis_name="c", subcore_axis_name="s")

    @functools.partial(
        pl.kernel,
        out_type=(jax.ShapeDtypeStruct((BATCH, D), jnp.float32),
                  jax.ShapeDtypeStruct((BATCH, D), jnp.float32)),
        mesh=mesh,
        scratch_types=[
            pltpu.VMEM((rows,), jnp.int32),
            pltpu.VMEM((rows, D), jnp.float32),
            pltpu.VMEM((rows, D), jnp.float32),
            pltpu.VMEM((rows, D), jnp.float32),
            pltpu.SemaphoreType.DMA,
        ],
        compiler_params=pltpu.CompilerParams(needs_layout_passes=False,
                                             use_tc_tiling_on_sc=False),
    )
    def fin(u0_hbm, u1_hbm, u2_hbm, i0_hbm, i1_hbm, i2_hbm, us_hbm, it_hbm,
            out_u, out_i, idx_v, g0, g1, g2, sem):
        cid = lax.axis_index("c")
        sid = lax.axis_index("s")
        base = sid * rows

        def side(idx_hbm, t0, t1, t2, out):
            pltpu.sync_copy(idx_hbm.at[pl.ds(base, rows)], idx_v)
            pltpu.async_copy(t0.at[idx_v], g0, sem).wait()
            pltpu.async_copy(t1.at[idx_v], g1, sem).wait()
            pltpu.async_copy(t2.at[idx_v], g2, sem).wait()

            def mean_body(e, _):
                for q in range(D // L):
                    sl = pl.ds(q * L, L)
                    g0[e, sl] = (g0[e, sl] + g1[e, sl] + g2[e, sl]) * (1.0 / 3.0)
                return 0

            lax.fori_loop(0, rows, mean_body, 0, unroll=2)
            pltpu.sync_copy(g0, out.at[pl.ds(base, rows)])

        @pl.when(cid == 0)
        def _():
            side(us_hbm, u0_hbm, u1_hbm, u2_hbm, out_u)

        @pl.when(cid == 1)
        def _():
            side(it_hbm, i0_hbm, i1_hbm, i2_hbm, out_i)

    return fin(u0, u1, u2, i0, i1, i2, users, pos_items)


def kernel(users, pos_items, user_emb, item_emb, adj_row, adj_col, adj_val):
    E = adj_row.shape[0] // 2
    r = adj_row[:E].astype(jnp.int32)
    ci = adj_col[:E].astype(jnp.int32) - N_USERS
    val = adj_val[:E]

    group = NS * CHUNK * 2 * G
    e_pad = ((E + group - 1) // group) * group
    pad = e_pad - E
    if pad:
        r = jnp.concatenate([r, jnp.zeros((pad,), jnp.int32)])
        ci = jnp.concatenate([ci, jnp.zeros((pad,), jnp.int32)])
        val = jnp.concatenate([val, jnp.zeros((pad,), jnp.float32)])
    nct = e_pad // (NS * CHUNK)
    edges = jnp.stack(
        [r.reshape(NS, nct, CHUNK), ci.reshape(NS, nct, CHUNK),
         jax.lax.bitcast_convert_type(val, jnp.int32).reshape(NS, nct, CHUNK)],
        axis=2)

    zeros = jnp.zeros((SLAB, D), jnp.float32)
    u1, i1 = _propagate(zeros, user_emb, item_emb, edges, nct)
    u2, i2 = _propagate(zeros, u1, i1, edges, nct)
    out_u, out_i = _finalize(user_emb, u1, u2, item_emb, i1, i2,
                             users.astype(jnp.int32), pos_items.astype(jnp.int32))
    return out_u, out_i


# speedup vs baseline: 1.3945x; 1.3945x over previous
import functools

import jax
import jax.numpy as jnp
from jax import lax
from jax.experimental import pallas as pl
from jax.experimental.pallas import tpu as pltpu
from jax.experimental.pallas import tpu_sc as plsc

N_USERS = 25000
N_ITEMS = 25000
D = 64
N_LAYERS = 2
BATCH = 2048

NC = 2
NS = 16
L = 16
CHUNK = 64
NBUF = 6
LAG = 3
G = 9
N_PAD = 25088
SLAB = N_PAD // NS


def _propagate(zeros, ego_u, ego_i, edges, nct):
    mesh = plsc.VectorSubcoreMesh(core_axis_name="c", subcore_axis_name="s")

    @functools.partial(
        pl.kernel,
        out_type=(jax.ShapeDtypeStruct((N_PAD, D), jnp.float32),
                  jax.ShapeDtypeStruct((N_PAD, D), jnp.float32)),
        mesh=mesh,
        scratch_types=(
            [pltpu.VMEM((G, 3, CHUNK), jnp.int32)] * 2
            + [pltpu.VMEM((CHUNK, D), jnp.float32)] * NBUF
            + [pltpu.VMEM_SHARED((N_PAD, D), jnp.float32)]
            + [pltpu.SemaphoreType.DMA] * (2 * NBUF + 2)
        ),
        compiler_params=pltpu.CompilerParams(needs_layout_passes=False,
                                             use_tc_tiling_on_sc=False),
    )
    def layer(zeros_hbm, ego_u_hbm, ego_i_hbm, edg_hbm,
              out_u, out_i, ibA, ibB, *rest):
        gbufs = rest[:NBUF]
        acc = rest[NBUF]
        gsems = rest[NBUF + 1:2 * NBUF + 1]
        ssems = rest[2 * NBUF + 1:3 * NBUF + 1]
        isems = rest[3 * NBUF + 1:3 * NBUF + 3]
        cid = lax.axis_index("c")
        sid = lax.axis_index("s")
        ibs = (ibA, ibB)

        pltpu.sync_copy(zeros_hbm, acc.at[pl.ds(sid * SLAB, SLAB)])
        plsc.subcore_barrier()

        ngr = nct // G
        nbody = nct // (2 * G)

        def side(flip, table):
            dr, sr = (0, 1) if flip == 0 else (1, 0)

            def sel(t):
                tb = 0 if (t % (2 * G)) < G else 1
                return tb, t % G, t % NBUF

            def g_start(t):
                tb, u, k = sel(t)
                pltpu.async_copy(table.at[ibs[tb].at[u, sr]], gbufs[k],
                                 gsems[k])

            def g_wait(t):
                tb, u, k = sel(t)
                pltpu.make_async_copy(table.at[ibs[tb].at[u, sr]], gbufs[k],
                                      gsems[k]).wait()

            def s_start(t):
                tb, u, k = sel(t)
                pltpu.async_copy(gbufs[k], acc.at[ibs[tb].at[u, dr]],
                                 ssems[k], add=True)

            def s_wait(t):
                tb, u, k = sel(t)
                pltpu.make_async_copy(gbufs[k], acc.at[ibs[tb].at[u, dr]],
                                      ssems[k]).wait()

            def i_start(tb, g):
                pltpu.async_copy(edg_hbm.at[sid, pl.ds(g * G, G)], ibs[tb],
                                 isems[tb])

            def i_wait(tb):
                pltpu.make_async_copy(edg_hbm.at[sid, pl.ds(0, G)], ibs[tb],
                                      isems[tb]).wait()

            def scale(t):
                tb, u, k = sel(t)
                buf = gbufs[k]

                def body(e, _):
                    vv = plsc.bitcast(
                        plsc.load_gather(
                            ibs[tb], [jnp.full((L,), u, jnp.int32),
                                      jnp.full((L,), 2, jnp.int32),
                                      jnp.full((L,), e, jnp.int32)]),
                        jnp.float32)
                    for q in range(D // L):
                        sl = pl.ds(q * L, L)
                        buf[e, sl] = buf[e, sl] * vv
                    return 0

                lax.fori_loop(0, CHUNK, body, 0, unroll=4)

            pltpu.sync_copy(edg_hbm.at[sid, pl.ds(0, G)], ibA)
            for t in range(LAG):
                g_start(t)

            def body(j, _):
                for t in range(2 * G):
                    g_wait(t)
                    scale(t)
                    s_start(t)
                    if t == LAG:
                        i_start(1, 2 * j + 1)
                    if t == G + LAG:
                        @pl.when(2 * j + 2 < ngr)
                        def _():
                            i_start(0, 2 * j + 2)
                    if t < LAG:
                        @pl.when(j > 0)
                        def _():
                            s_wait(t - LAG)
                    else:
                        s_wait(t - LAG)
                    if t == G - LAG:
                        i_wait(1)
                    if t == 2 * G - LAG:
                        @pl.when(2 * j + 2 < ngr)
                        def _():
                            i_wait(0)
                    if t < 2 * G - LAG:
                        g_start(t + LAG)
                    else:
                        @pl.when(2 * j + 2 < ngr)
                        def _():
                            g_start(t + LAG)
                return 0

            lax.fori_loop(0, nbody, body, 0)
            for t in range(LAG):
                s_wait(2 * G - LAG + t)

        @pl.when(cid == 0)
        def _():
            side(0, ego_i_hbm)

        @pl.when(cid == 1)
        def _():
            side(1, ego_u_hbm)

        plsc.subcore_barrier()
        sl = pl.ds(sid * SLAB, SLAB)

        @pl.when(cid == 0)
        def _():
            pltpu.sync_copy(acc.at[sl], out_u.at[sl])

        @pl.when(cid == 1)
        def _():
            pltpu.sync_copy(acc.at[sl], out_i.at[sl])

    return layer(zeros, ego_u, ego_i, edges)


def _finalize(u0, u1, u2, i0, i1, i2, users, pos_items):
    rows = BATCH // NS
    mesh = plsc.VectorSubcoreMesh(core_axis_name="c", subcore_axis_name="s")

    @functools.partial(
        pl.kernel,
        out_type=(jax.ShapeDtypeStruct((BATCH, D), jnp.float32),
                  jax.ShapeDtypeStruct((BATCH, D), jnp.float32)),
        mesh=mesh,
        scratch_types=[
            pltpu.VMEM((rows,), jnp.int32),
            pltpu.VMEM((rows, D), jnp.float32),
            pltpu.VMEM((rows, D), jnp.float32),
            pltpu.VMEM((rows, D), jnp.float32),
            pltpu.SemaphoreType.DMA,
        ],
        compiler_params=pltpu.CompilerParams(needs_layout_passes=False,
                                             use_tc_tiling_on_sc=False),
    )
    def fin(u0_hbm, u1_hbm, u2_hbm, i0_hbm, i1_hbm, i2_hbm, us_hbm, it_hbm,
            out_u, out_i, idx_v, g0, g1, g2, sem):
        cid = lax.axis_index("c")
        sid = lax.axis_index("s")
        base = sid * rows

        def side(idx_hbm, t0, t1, t2, out):
            pltpu.sync_copy(idx_hbm.at[pl.ds(base, rows)], idx_v)
            pltpu.async_copy(t0.at[idx_v], g0, sem).wait()
            pltpu.async_copy(t1.at[idx_v], g1, sem).wait()
            pltpu.async_copy(t2.at[idx_v], g2, sem).wait()

            def mean_body(e, _):
                for q in range(D // L):
                    sl = pl.ds(q * L, L)
                    g0[e, sl] = (g0[e, sl] + g1[e, sl] + g2[e, sl]) * (1.0 / 3.0)
                return 0

            lax.fori_loop(0, rows, mean_body, 0, unroll=2)
            pltpu.sync_copy(g0, out.at[pl.ds(base, rows)])

        @pl.when(cid == 0)
        def _():
            side(us_hbm, u0_hbm, u1_hbm, u2_hbm, out_u)

        @pl.when(cid == 1)
        def _():
            side(it_hbm, i0_hbm, i1_hbm, i2_hbm, out_i)

    return fin(u0, u1, u2, i0, i1, i2, users, pos_items)


def kernel(users, pos_items, user_emb, item_emb, adj_row, adj_col, adj_val):
    E = adj_row.shape[0] // 2
    r = adj_row[:E].astype(jnp.int32)
    ci = adj_col[:E].astype(jnp.int32) - N_USERS
    val = adj_val[:E]

    group = NS * CHUNK * 2 * G
    e_pad = ((E + group - 1) // group) * group
    pad = e_pad - E
    if pad:
        r = jnp.concatenate([r, jnp.zeros((pad,), jnp.int32)])
        ci = jnp.concatenate([ci, jnp.zeros((pad,), jnp.int32)])
        val = jnp.concatenate([val, jnp.zeros((pad,), jnp.float32)])
    nct = e_pad // (NS * CHUNK)
    edges = jnp.stack(
        [r.reshape(NS, nct, CHUNK), ci.reshape(NS, nct, CHUNK),
         jax.lax.bitcast_convert_type(val, jnp.int32).reshape(NS, nct, CHUNK)],
        axis=2)

    zeros = jnp.zeros((SLAB, D), jnp.float32)
    u1, i1 = _propagate(zeros, user_emb, item_emb, edges, nct)
    u2, i2 = _propagate(zeros, u1, i1, edges, nct)
    out_u, out_i = _finalize(user_emb, u1, u2, item_emb, i1, i2,
                             users.astype(jnp.int32), pos_items.astype(jnp.int32))
    return out_u, out_i
